# baseline (speedup 1.0000x reference)
import functools

import jax
import jax.numpy as jnp
from jax import lax
from jax.experimental import pallas as pl
from jax.experimental.pallas import tpu as pltpu
from jax.experimental.pallas import tpu_sc as plsc

E = 16 * 2048
NC, NS = 2, 16
NW = NC * NS
PER_W = E // NW
NCH = 8
CH = PER_W // NCH
PHONE_D, MIDI_D = 128, 64

BLK = 2048


def _sc_body(ph_idx_hbm, md_idx_hbm, pt_hbm, mt_hbm, out_ph_hbm, out_md_hbm,
             idx_ph_v, idx_md_v, ph_rows, md_rows, sem_p, sem_m):
    wid = lax.axis_index("s") * NC + lax.axis_index("c")
    base = wid * PER_W

    pltpu.sync_copy(ph_idx_hbm.at[wid], idx_ph_v)
    pltpu.sync_copy(md_idx_hbm.at[wid], idx_md_v)

    gathers_p = []
    gathers_m = []
    for c in range(NCH):
        gathers_p.append(pltpu.make_async_copy(
            pt_hbm.at[idx_ph_v.at[c]], ph_rows.at[c % 2], sem_p))
        gathers_m.append(pltpu.make_async_copy(
            mt_hbm.at[idx_md_v.at[c]], md_rows.at[c % 2], sem_m))

    gathers_p[0].start()
    gathers_m[0].start()
    for c in range(NCH):
        gathers_p[c].wait()
        gathers_m[c].wait()
        row0 = base + c * CH
        if c + 1 < NCH:
            gathers_p[c + 1].start()
            gathers_m[c + 1].start()
        pltpu.sync_copy(ph_rows.at[c % 2], out_ph_hbm.at[pl.ds(row0, CH)])
        pltpu.sync_copy(md_rows.at[c % 2], out_md_hbm.at[pl.ds(row0, CH)])


@jax.jit
def _sc_lookup(ph_idx, md_idx, phone_table, midi_table):
    mesh = plsc.VectorSubcoreMesh(core_axis_name="c", subcore_axis_name="s")
    return pl.kernel(
        _sc_body,
        out_type=[jax.ShapeDtypeStruct((E, PHONE_D), jnp.float32),
                  jax.ShapeDtypeStruct((E, MIDI_D), jnp.float32)],
        mesh=mesh,
        scratch_types=[
            pltpu.VMEM((NCH, CH), jnp.int32),
            pltpu.VMEM((NCH, CH), jnp.int32),
            pltpu.VMEM((2, CH, PHONE_D), jnp.float32),
            pltpu.VMEM((2, CH, MIDI_D), jnp.float32),
            pltpu.SemaphoreType.DMA,
            pltpu.SemaphoreType.DMA,
        ],
        compiler_params=pltpu.CompilerParams(use_tc_tiling_on_sc=False),
    )(ph_idx, md_idx, phone_table, midi_table)


def _tc_body(f0_ref, un_ref, wf_ref, bf_ref, wu_ref, bu_ref, of_ref, ou_ref):
    of_ref[...] = f0_ref[...] * wf_ref[...] + bf_ref[...]
    ou_ref[...] = un_ref[...] * wu_ref[...] + bu_ref[...]


@jax.jit
def _tc_project(f0r, unr, wf, bf, wu, bu):
    grid = E // BLK
    blk_in = pl.BlockSpec((BLK, 1), lambda i: (i, 0))
    full = lambda shape: pl.BlockSpec(shape, lambda i: (0, 0))
    out_spec = lambda d: pl.BlockSpec((BLK, d), lambda i: (i, 0))
    return pl.pallas_call(
        _tc_body,
        grid=(grid,),
        in_specs=[blk_in, blk_in, full((1, 64)), full((1, 64)),
                  full((1, 16)), full((1, 16))],
        out_specs=[out_spec(64), out_spec(16)],
        out_shape=[jax.ShapeDtypeStruct((E, 64), jnp.float32),
                   jax.ShapeDtypeStruct((E, 16), jnp.float32)],
    )(f0r, unr, wf, bf, wu, bu)


def kernel(f0, phone_label, phone_duration, midi_label, unvoiced_flag,
           W_f0, b_f0, phone_table, midi_table, W_unv, b_unv):
    B, S = phone_label.shape
    ph_idx = phone_label.astype(jnp.int32).reshape(NW, NCH, CH)
    md_idx = midi_label.astype(jnp.int32).reshape(NW, NCH, CH)
    f0r = f0.reshape(E, 1)
    unr = unvoiced_flag.reshape(E, 1)
    wf = W_f0.reshape(1, -1)
    bf = b_f0.reshape(1, -1)
    wu = W_unv.reshape(1, -1)
    bu = b_unv.reshape(1, -1)

    op, om = _sc_lookup(ph_idx, md_idx, phone_table, midi_table)
    of, ou = _tc_project(f0r, unr, wf, bf, wu, bu)
    return (of.reshape(B, S, 64), op.reshape(B, S, PHONE_D),
            om.reshape(B, S, MIDI_D), ou.reshape(B, S, 16))
